# baseline (speedup 1.0000x reference)
import functools

import jax
import jax.numpy as jnp
from jax import lax
from jax.experimental import pallas as pl
from jax.experimental.pallas import tpu as pltpu
from jax.experimental.pallas import tpu_sc as plsc

EMBED = 64
EPS = 1e-12
LANES = 16
CHUNK = 128


def _make_sc_gather(n_tokens):
    mesh = plsc.VectorSubcoreMesh(core_axis_name="c", subcore_axis_name="s")
    info = plsc.get_sparse_core_info()
    nw = info.num_cores * info.num_subcores
    assert n_tokens % (nw * 2 * CHUNK) == 0
    tok_per_w = n_tokens // nw
    n_chunks = tok_per_w // CHUNK

    @functools.partial(
        pl.kernel,
        mesh=mesh,
        compiler_params=pltpu.CompilerParams(
            needs_layout_passes=False, use_tc_tiling_on_sc=False
        ),
        out_type=jax.ShapeDtypeStruct((n_tokens, EMBED), jnp.float32),
        scratch_types=[
            pltpu.SMEM((2, CHUNK), jnp.int32),
            pltpu.VMEM_SHARED((16, 2, CHUNK, EMBED), jnp.float32),
            pltpu.VMEM_SHARED((16, 2, CHUNK), jnp.int32),
            pltpu.SemaphoreType.DMA,
            pltpu.SemaphoreType.DMA,
            pltpu.SemaphoreType.DMA,
            pltpu.SemaphoreType.DMA,
        ],
    )
    def body(ids_hbm, tok_hbm, out_hbm, idsv, sprows, spids, g0, g1, o0, o1):
        sid = lax.axis_index("s")
        wid = sid * info.num_cores + lax.axis_index("c")
        w_base = wid * tok_per_w
        gsems = (g0, g1)
        osems = (o0, o1)

        def stage_ids(c, b):
            base = w_base + c * CHUNK
            pltpu.sync_copy(ids_hbm.at[pl.ds(base, CHUNK)], spids.at[sid, b])
            pltpu.sync_copy(spids.at[sid, b], idsv.at[b])

        def fire_rows(b):
            my_rows = sprows.at[sid, b]

            def one(t, _):
                rid = idsv[b, t]
                pltpu.make_async_copy(
                    tok_hbm.at[pl.ds(rid, 1)],
                    my_rows.at[pl.ds(t, 1)],
                    gsems[b],
                ).start()
                return 0

            lax.fori_loop(0, CHUNK, one, 0)

        def drain_rows(b):
            pltpu.make_async_copy(
                tok_hbm.at[pl.ds(0, CHUNK)], sprows.at[sid, b], gsems[b]
            ).wait()

        def start_out(c, b):
            base = w_base + c * CHUNK
            pltpu.make_async_copy(
                sprows.at[sid, b], out_hbm.at[pl.ds(base, CHUNK)], osems[b]
            ).start()

        def wait_out(c, b):
            base = w_base + c * CHUNK
            pltpu.make_async_copy(
                sprows.at[sid, b], out_hbm.at[pl.ds(base, CHUNK)], osems[b]
            ).wait()

        stage_ids(0, 0)
        fire_rows(0)

        def do_pair(i, _):
            for b in (0, 1):
                c = 2 * i + b
                bn = 1 - b

                @pl.when(c + 1 < n_chunks)
                def _():
                    stage_ids(c + 1, bn)

                    @pl.when(c >= 1)
                    def _():
                        wait_out(c - 1, bn)

                    fire_rows(bn)

                drain_rows(b)
                start_out(c, b)
            return 0

        lax.fori_loop(0, n_chunks // 2, do_pair, 0)
        wait_out(n_chunks - 2, 0)
        wait_out(n_chunks - 1, 1)

    return body


def _tc_ln_body(tok_ref, pos_ref, typ_ref, gamma_ref, beta_ref, out_ref):
    x = tok_ref[...]
    comb = pos_ref[...] + typ_ref[0][None, :]
    x = x + comb[None, :, :]
    mean = jnp.mean(x, axis=-1, keepdims=True)
    var = jnp.mean(jnp.square(x - mean), axis=-1, keepdims=True)
    y = (x - mean) * lax.rsqrt(var + EPS)
    out_ref[...] = (
        y * gamma_ref[...][None, None, :] + beta_ref[...][None, None, :]
    )


def _make_tc_ln(batch, seq_len, block_b):
    grid = (batch // block_b,)
    return pl.pallas_call(
        _tc_ln_body,
        grid=grid,
        in_specs=[
            pl.BlockSpec((block_b, seq_len, EMBED), lambda i: (i, 0, 0)),
            pl.BlockSpec((seq_len, EMBED), lambda i: (0, 0)),
            pl.BlockSpec((2, EMBED), lambda i: (0, 0)),
            pl.BlockSpec((EMBED,), lambda i: (0,)),
            pl.BlockSpec((EMBED,), lambda i: (0,)),
        ],
        out_specs=pl.BlockSpec((block_b, seq_len, EMBED), lambda i: (i, 0, 0)),
        out_shape=jax.ShapeDtypeStruct((batch, seq_len, EMBED), jnp.float32),
    )


def kernel(input_ids, token_table, pos_table, type_table, gamma, beta):
    batch, seq_len = input_ids.shape
    n_tokens = batch * seq_len
    ids = input_ids.reshape(-1).astype(jnp.int32)
    gathered = _make_sc_gather(n_tokens)(ids, token_table)
    tok_emb = gathered.reshape(batch, seq_len, EMBED)
    ln = _make_tc_ln(batch, seq_len, block_b=8)
    return ln(tok_emb, pos_table[:seq_len], type_table, gamma, beta)


# speedup vs baseline: 1.0510x; 1.0510x over previous
import functools

import jax
import jax.numpy as jnp
from jax import lax
from jax.experimental import pallas as pl
from jax.experimental.pallas import tpu as pltpu
from jax.experimental.pallas import tpu_sc as plsc

EMBED = 64
EPS = 1e-12
LANES = 16
CHUNK = 128


def _make_sc_gather(n_tokens):
    mesh = plsc.VectorSubcoreMesh(core_axis_name="c", subcore_axis_name="s")
    info = plsc.get_sparse_core_info()
    nw = info.num_cores * info.num_subcores
    assert n_tokens % (nw * 2 * CHUNK) == 0
    tok_per_w = n_tokens // nw
    n_chunks = tok_per_w // CHUNK

    @functools.partial(
        pl.kernel,
        mesh=mesh,
        compiler_params=pltpu.CompilerParams(
            needs_layout_passes=False, use_tc_tiling_on_sc=False
        ),
        out_type=jax.ShapeDtypeStruct((n_tokens, EMBED), jnp.float32),
        scratch_types=[
            pltpu.SMEM((2, CHUNK), jnp.int32),
            pltpu.VMEM_SHARED((16, 2, CHUNK, EMBED), jnp.float32),
            pltpu.VMEM_SHARED((16, 2, CHUNK), jnp.int32),
            pltpu.SemaphoreType.DMA,
            pltpu.SemaphoreType.DMA,
            pltpu.SemaphoreType.DMA,
            pltpu.SemaphoreType.DMA,
        ],
    )
    def body(ids_hbm, tok_hbm, out_hbm, idsv, sprows, spids, g0, g1, o0, o1):
        sid = lax.axis_index("s")
        wid = sid * info.num_cores + lax.axis_index("c")
        w_base = wid * tok_per_w
        gsems = (g0, g1)
        osems = (o0, o1)

        def stage_ids(c, b):
            base = w_base + c * CHUNK
            pltpu.sync_copy(ids_hbm.at[pl.ds(base, CHUNK)], spids.at[sid, b])
            pltpu.sync_copy(spids.at[sid, b], idsv.at[b])

        def fire_rows(b):
            my_rows = sprows.at[sid, b]

            def one(i, _):
                for j in range(4):
                    t = i * 4 + j
                    rid = idsv[b, t]
                    pltpu.make_async_copy(
                        tok_hbm.at[pl.ds(rid, 1)],
                        my_rows.at[pl.ds(t, 1)],
                        gsems[b],
                    ).start()
                return 0

            lax.fori_loop(0, CHUNK // 4, one, 0)

        def drain_rows(b):
            pltpu.make_async_copy(
                tok_hbm.at[pl.ds(0, CHUNK)], sprows.at[sid, b], gsems[b]
            ).wait()

        def start_out(c, b):
            base = w_base + c * CHUNK
            pltpu.make_async_copy(
                sprows.at[sid, b], out_hbm.at[pl.ds(base, CHUNK)], osems[b]
            ).start()

        def wait_out(c, b):
            base = w_base + c * CHUNK
            pltpu.make_async_copy(
                sprows.at[sid, b], out_hbm.at[pl.ds(base, CHUNK)], osems[b]
            ).wait()

        stage_ids(0, 0)
        fire_rows(0)

        def do_pair(i, _):
            for b in (0, 1):
                c = 2 * i + b
                bn = 1 - b

                @pl.when(c + 1 < n_chunks)
                def _():
                    stage_ids(c + 1, bn)

                    @pl.when(c >= 1)
                    def _():
                        wait_out(c - 1, bn)

                    fire_rows(bn)

                drain_rows(b)
                start_out(c, b)
            return 0

        lax.fori_loop(0, n_chunks // 2, do_pair, 0)
        wait_out(n_chunks - 2, 0)
        wait_out(n_chunks - 1, 1)

    return body


def _tc_ln_body(tok_ref, pos_ref, typ_ref, gamma_ref, beta_ref, out_ref):
    x = tok_ref[...]
    comb = pos_ref[...] + typ_ref[0][None, :]
    x = x + comb[None, :, :]
    mean = jnp.mean(x, axis=-1, keepdims=True)
    var = jnp.mean(jnp.square(x - mean), axis=-1, keepdims=True)
    y = (x - mean) * lax.rsqrt(var + EPS)
    out_ref[...] = (
        y * gamma_ref[...][None, None, :] + beta_ref[...][None, None, :]
    )


def _make_tc_ln(batch, seq_len, block_b):
    grid = (batch // block_b,)
    return pl.pallas_call(
        _tc_ln_body,
        grid=grid,
        in_specs=[
            pl.BlockSpec((block_b, seq_len, EMBED), lambda i: (i, 0, 0)),
            pl.BlockSpec((seq_len, EMBED), lambda i: (0, 0)),
            pl.BlockSpec((2, EMBED), lambda i: (0, 0)),
            pl.BlockSpec((EMBED,), lambda i: (0,)),
            pl.BlockSpec((EMBED,), lambda i: (0,)),
        ],
        out_specs=pl.BlockSpec((block_b, seq_len, EMBED), lambda i: (i, 0, 0)),
        out_shape=jax.ShapeDtypeStruct((batch, seq_len, EMBED), jnp.float32),
    )


def kernel(input_ids, token_table, pos_table, type_table, gamma, beta):
    batch, seq_len = input_ids.shape
    n_tokens = batch * seq_len
    ids = input_ids.reshape(-1).astype(jnp.int32)
    gathered = _make_sc_gather(n_tokens)(ids, token_table)
    tok_emb = gathered.reshape(batch, seq_len, EMBED)
    ln = _make_tc_ln(batch, seq_len, block_b=64)
    return ln(tok_emb, pos_table[:seq_len], type_table, gamma, beta)
